# baseline (speedup 1.0000x reference)
import functools

import jax
import jax.numpy as jnp
from jax import lax
from jax.experimental import pallas as pl
from jax.experimental.pallas import tpu as pltpu
from jax.experimental.pallas import tpu_sc as plsc

Q = 1024
D_CAT = 1536
D_HID = 768
D_OUT = 384
K_TOT = 100000
K_TOP = 3
KB = 2000
N_BLK = K_TOT // KB
R_BLK = KB // 8


def _mlp_body(text_ref, image_ref, w1a_ref, w1b_ref, b1_ref, w2_ref, b2_ref,
              proj_ref):
    h = jnp.dot(text_ref[...], w1a_ref[...], preferred_element_type=jnp.float32)
    h = h + jnp.dot(image_ref[...], w1b_ref[...],
                    preferred_element_type=jnp.float32)
    h = jnp.maximum(h + b1_ref[...], 0.0)
    m2 = -2.0 * (jnp.dot(h, w2_ref[...], preferred_element_type=jnp.float32)
                 + b2_ref[...])
    phi = m2.astype(jnp.bfloat16)
    plo = (m2 - phi.astype(jnp.float32)).astype(jnp.bfloat16)
    proj_ref[...] = jnp.concatenate([phi, phi, plo], axis=1)


def _mlp(text_emb, image_emb, w1a, w1b, b1, w2, b2):
    return pl.pallas_call(
        _mlp_body,
        out_shape=jax.ShapeDtypeStruct((Q, 3 * D_OUT), jnp.bfloat16),
    )(text_emb, image_emb, w1a, w1b, b1, w2, b2)


def _insert3(c, ic, v0, v1, v2, i0, i1, i2):
    lt0, lt1, lt2 = c < v0, c < v1, c < v2
    nv0 = jnp.minimum(v0, c)
    nv1 = jnp.where(lt1, jnp.maximum(v0, c), v1)
    nv2 = jnp.where(lt2, jnp.maximum(v1, c), v2)
    ni0 = jnp.where(lt0, ic, i0)
    ni1 = jnp.where(lt1, jnp.where(lt0, i0, ic), i1)
    ni2 = jnp.where(lt2, jnp.where(lt1, i1, ic), i2)
    return nv0, nv1, nv2, ni0, ni1, ni2


def _ksq_body(keys_ref, ksq_ref):
    kb = keys_ref[...]
    ksq_ref[...] = jnp.sum(kb * kb, axis=1, keepdims=True)


def _ksq(keys):
    return pl.pallas_call(
        _ksq_body,
        grid=(N_BLK,),
        in_specs=[pl.BlockSpec((KB, D_OUT), lambda b: (b, 0))],
        out_specs=pl.BlockSpec((KB, 1), lambda b: (b, 0)),
        out_shape=jax.ShapeDtypeStruct((K_TOT, 1), jnp.float32),
    )(keys)


_GRPS = 8
_QG = Q // _GRPS


def _topk_body(keys_ref, ksq_ref, proj_ref, out_val_ref, out_idx_ref, s_scr,
               bval, bidx):
    b = pl.program_id(0)

    @pl.when(b == 0)
    def _init():
        bval[...] = jnp.full_like(bval, jnp.inf)
        bidx[...] = jnp.zeros_like(bidx)

    kb = keys_ref[...]
    khi = kb.astype(jnp.bfloat16)
    klo = (kb - khi.astype(jnp.float32)).astype(jnp.bfloat16)
    kcat = jnp.concatenate([khi, klo, khi], axis=1)
    s = lax.dot_general(kcat, proj_ref[...], (((1,), (0,)), ((), ())),
                        preferred_element_type=jnp.float32)
    s_scr[...] = s + ksq_ref[...]

    idx0 = b * KB
    sub_iota = jax.lax.broadcasted_iota(jnp.int32, (8, _QG), 0)
    v_parts = [[], [], []]
    i_parts = [[], [], []]
    for g in range(_GRPS):
        init = (jnp.full((8, _QG), jnp.inf, jnp.float32),) * 3 + (
            jnp.zeros((8, _QG), jnp.int32),) * 3

        def fold(r, st, g=g):
            r0 = r * 80
            for j in range(10):
                c = s_scr[pl.ds(pl.multiple_of(r0 + j * 8, 8), 8),
                          pl.ds(g * _QG, _QG)]
                ic = sub_iota + (idx0 + r0 + j * 8)
                st = _insert3(c, ic, *st)
            return st

        res = lax.fori_loop(0, R_BLK // 10, fold, init)
        for t in range(3):
            v_parts[t].append(res[t])
            i_parts[t].append(res[3 + t])

    v0, v1, v2 = (jnp.concatenate(p, axis=1) for p in v_parts)
    i0, i1, i2 = (jnp.concatenate(p, axis=1) for p in i_parts)

    stack = jnp.concatenate([v0, v1, v2], axis=0)
    istack = jnp.concatenate([i0, i1, i2], axis=0)
    for _ in range(K_TOP):
        m = jnp.min(stack, axis=0, keepdims=True)
        cand = jnp.where(stack == m, istack, jnp.int32(2**31 - 1))
        ci = jnp.min(cand, axis=0, keepdims=True)
        stack = jnp.where(istack == ci, jnp.inf, stack)
        w0, w1, w2, j0, j1, j2 = _insert3(
            m, ci, bval[0:1], bval[1:2], bval[2:3],
            bidx[0:1], bidx[1:2], bidx[2:3])
        bval[0:1], bval[1:2], bval[2:3] = w0, w1, w2
        bidx[0:1], bidx[1:2], bidx[2:3] = j0, j1, j2

    @pl.when(b == pl.num_programs(0) - 1)
    def _emit():
        out_val_ref[...] = bval[...]
        out_idx_ref[...] = bidx[...]


def _topk(keys, ksq, proj):
    return pl.pallas_call(
        _topk_body,
        grid=(N_BLK,),
        in_specs=[
            pl.BlockSpec((KB, D_OUT), lambda b: (b, 0)),
            pl.BlockSpec((KB, 1), lambda b: (b, 0)),
            pl.BlockSpec((3 * D_OUT, Q), lambda b: (0, 0)),
        ],
        out_specs=[
            pl.BlockSpec((8, Q), lambda b: (0, 0)),
            pl.BlockSpec((8, Q), lambda b: (0, 0)),
        ],
        out_shape=[
            jax.ShapeDtypeStruct((8, Q), jnp.float32),
            jax.ShapeDtypeStruct((8, Q), jnp.int32),
        ],
        scratch_shapes=[
            pltpu.VMEM((KB, Q), jnp.float32),
            pltpu.VMEM((8, Q), jnp.float32),
            pltpu.VMEM((8, Q), jnp.int32),
        ],
    )(keys, ksq, proj)


_NC, _NS = 2, 16
_NW = _NC * _NS
_B_TOT = Q * K_TOP
_BPW = _B_TOT // _NW


def _sc_gather_body(table_hbm, idx_hbm, out_hbm, idx_v, rows_v, sem):
    wid = lax.axis_index("s") * _NC + lax.axis_index("c")
    base = wid * _BPW
    pltpu.sync_copy(idx_hbm.at[pl.ds(base, _BPW)], idx_v)
    pltpu.async_copy(table_hbm.at[idx_v], rows_v, sem).wait()
    pltpu.sync_copy(rows_v, out_hbm.at[pl.ds(base, _BPW)])


def _sc_gather(keys, flat_idx):
    mesh = plsc.VectorSubcoreMesh(core_axis_name="c", subcore_axis_name="s")
    k = functools.partial(
        pl.kernel, mesh=mesh,
        out_type=jax.ShapeDtypeStruct((_B_TOT, D_OUT), jnp.float32),
        scratch_types=[
            pltpu.VMEM((_BPW,), jnp.int32),
            pltpu.VMEM((_BPW, D_OUT), jnp.float32),
            pltpu.SemaphoreType.DMA,
        ],
    )(_sc_gather_body)
    return k(keys, flat_idx)


def kernel(text_emb, image_emb, keys, W1, b1, W2, b2):
    pcat = _mlp(text_emb, image_emb, W1[:D_HID], W1[D_HID:],
                b1.reshape(1, D_HID), W2, b2.reshape(1, D_OUT))
    _, idx8 = _topk(keys, _ksq(keys), pcat.T)
    I = idx8[:K_TOP].T
    retrieved = _sc_gather(keys, I.reshape(-1)).reshape(Q, K_TOP, D_OUT)
    return retrieved, I


# speedup vs baseline: 1.2354x; 1.2354x over previous
import functools

import jax
import jax.numpy as jnp
from jax import lax
from jax.experimental import pallas as pl
from jax.experimental.pallas import tpu as pltpu
from jax.experimental.pallas import tpu_sc as plsc

Q = 1024
D_CAT = 1536
D_HID = 768
D_OUT = 384
K_TOT = 100000
K_TOP = 3
KB = 2000
N_BLK = K_TOT // KB
R_BLK = KB // 8


def _mlp_body(text_ref, image_ref, w1a_ref, w1b_ref, b1_ref, w2_ref, b2_ref,
              proj_ref):
    h = jnp.dot(text_ref[...], w1a_ref[...], preferred_element_type=jnp.float32)
    h = h + jnp.dot(image_ref[...], w1b_ref[...],
                    preferred_element_type=jnp.float32)
    h = jnp.maximum(h + b1_ref[...], 0.0)
    proj_ref[...] = -2.0 * (
        jnp.dot(h, w2_ref[...], preferred_element_type=jnp.float32)
        + b2_ref[...])


def _mlp(text_emb, image_emb, w1a, w1b, b1, w2, b2):
    return pl.pallas_call(
        _mlp_body,
        out_shape=jax.ShapeDtypeStruct((Q, D_OUT), jnp.float32),
    )(text_emb, image_emb, w1a, w1b, b1, w2, b2)


def _insert3(c, ic, v0, v1, v2, i0, i1, i2):
    lt0, lt1, lt2 = c < v0, c < v1, c < v2
    nv0 = jnp.minimum(v0, c)
    nv1 = jnp.where(lt1, jnp.maximum(v0, c), v1)
    nv2 = jnp.where(lt2, jnp.maximum(v1, c), v2)
    ni0 = jnp.where(lt0, ic, i0)
    ni1 = jnp.where(lt1, jnp.where(lt0, i0, ic), i1)
    ni2 = jnp.where(lt2, jnp.where(lt1, i1, ic), i2)
    return nv0, nv1, nv2, ni0, ni1, ni2


def _ksq_body(keys_ref, ksq_ref):
    kb = keys_ref[...]
    ksq_ref[...] = jnp.sum(kb * kb, axis=1, keepdims=True)


def _ksq(keys):
    return pl.pallas_call(
        _ksq_body,
        grid=(N_BLK,),
        in_specs=[pl.BlockSpec((KB, D_OUT), lambda b: (b, 0))],
        out_specs=pl.BlockSpec((KB, 1), lambda b: (b, 0)),
        out_shape=jax.ShapeDtypeStruct((K_TOT, 1), jnp.float32),
    )(keys)


_GRPS = 8
_QG = Q // _GRPS


def _topk_body(keys_ref, ksq_ref, proj_ref, out_val_ref, out_idx_ref, s_scr,
               bval, bidx):
    b = pl.program_id(0)

    @pl.when(b == 0)
    def _init():
        bval[...] = jnp.full_like(bval, jnp.inf)
        bidx[...] = jnp.zeros_like(bidx)

    kb = keys_ref[...]
    s = lax.dot_general(kb, proj_ref[...], (((1,), (1,)), ((), ())),
                        preferred_element_type=jnp.float32)
    s_scr[...] = s + ksq_ref[...]

    base = jax.lax.broadcasted_iota(jnp.int32, (8, Q), 0) + b * KB
    init = (jnp.full((8, Q), jnp.inf, jnp.float32),) * 3 + (
        jnp.zeros((8, Q), jnp.int32),) * 3

    def fold(r, st):
        c = s_scr[pl.ds(pl.multiple_of(r * 8, 8), 8), :]
        ic = base + r * 8
        return _insert3(c, ic, *st)

    v0, v1, v2, i0, i1, i2 = lax.fori_loop(0, R_BLK, fold, init, unroll=2)

    stack = jnp.concatenate([v0, v1, v2], axis=0)
    istack = jnp.concatenate([i0, i1, i2], axis=0)
    for _ in range(K_TOP):
        m = jnp.min(stack, axis=0, keepdims=True)
        cand = jnp.where(stack == m, istack, jnp.int32(2**31 - 1))
        ci = jnp.min(cand, axis=0, keepdims=True)
        stack = jnp.where(istack == ci, jnp.inf, stack)
        w0, w1, w2, j0, j1, j2 = _insert3(
            m, ci, bval[0:1], bval[1:2], bval[2:3],
            bidx[0:1], bidx[1:2], bidx[2:3])
        bval[0:1], bval[1:2], bval[2:3] = w0, w1, w2
        bidx[0:1], bidx[1:2], bidx[2:3] = j0, j1, j2

    @pl.when(b == pl.num_programs(0) - 1)
    def _emit():
        out_val_ref[...] = bval[...]
        out_idx_ref[...] = bidx[...]


def _topk(keys, ksq, proj):
    return pl.pallas_call(
        _topk_body,
        grid=(N_BLK,),
        in_specs=[
            pl.BlockSpec((KB, D_OUT), lambda b: (b, 0)),
            pl.BlockSpec((KB, 1), lambda b: (b, 0)),
            pl.BlockSpec((Q, D_OUT), lambda b: (0, 0)),
        ],
        out_specs=[
            pl.BlockSpec((8, Q), lambda b: (0, 0)),
            pl.BlockSpec((8, Q), lambda b: (0, 0)),
        ],
        out_shape=[
            jax.ShapeDtypeStruct((8, Q), jnp.float32),
            jax.ShapeDtypeStruct((8, Q), jnp.int32),
        ],
        scratch_shapes=[
            pltpu.VMEM((KB, Q), jnp.float32),
            pltpu.VMEM((8, Q), jnp.float32),
            pltpu.VMEM((8, Q), jnp.int32),
        ],
    )(keys, ksq, proj)


_NC, _NS = 2, 16
_NW = _NC * _NS
_B_TOT = Q * K_TOP
_BPW = _B_TOT // _NW


def _sc_gather_body(table_hbm, idx_hbm, out_hbm, idx_v, rows_v, sem):
    wid = lax.axis_index("s") * _NC + lax.axis_index("c")
    base = wid * _BPW
    pltpu.sync_copy(idx_hbm.at[pl.ds(base, _BPW)], idx_v)
    pltpu.async_copy(table_hbm.at[idx_v], rows_v, sem).wait()
    pltpu.sync_copy(rows_v, out_hbm.at[pl.ds(base, _BPW)])


def _sc_gather(keys, flat_idx):
    mesh = plsc.VectorSubcoreMesh(core_axis_name="c", subcore_axis_name="s")
    k = functools.partial(
        pl.kernel, mesh=mesh,
        out_type=jax.ShapeDtypeStruct((_B_TOT, D_OUT), jnp.float32),
        scratch_types=[
            pltpu.VMEM((_BPW,), jnp.int32),
            pltpu.VMEM((_BPW, D_OUT), jnp.float32),
            pltpu.SemaphoreType.DMA,
        ],
    )(_sc_gather_body)
    return k(keys, flat_idx)


def kernel(text_emb, image_emb, keys, W1, b1, W2, b2):
    proj = _mlp(text_emb, image_emb, W1[:D_HID], W1[D_HID:],
                b1.reshape(1, D_HID), W2, b2.reshape(1, D_OUT))
    _, idx8 = _topk(keys, _ksq(keys), proj)
    I = idx8[:K_TOP].T
    retrieved = _sc_gather(keys, I.reshape(-1)).reshape(Q, K_TOP, D_OUT)
    return retrieved, I


# speedup vs baseline: 1.4178x; 1.1476x over previous
import functools

import jax
import jax.numpy as jnp
from jax import lax
from jax.experimental import pallas as pl
from jax.experimental.pallas import tpu as pltpu
from jax.experimental.pallas import tpu_sc as plsc

Q = 1024
D_CAT = 1536
D_HID = 768
D_OUT = 384
K_TOT = 100000
K_TOP = 3
KB = 2000
N_BLK = K_TOT // KB
R_BLK = KB // 8


def _mlp_body(text_ref, image_ref, w1a_ref, w1b_ref, b1_ref, w2_ref, b2_ref,
              proj_ref):
    h = jnp.dot(text_ref[...], w1a_ref[...], preferred_element_type=jnp.float32)
    h = h + jnp.dot(image_ref[...], w1b_ref[...],
                    preferred_element_type=jnp.float32)
    h = jnp.maximum(h + b1_ref[...], 0.0)
    proj_ref[...] = -2.0 * (
        jnp.dot(h, w2_ref[...], preferred_element_type=jnp.float32)
        + b2_ref[...])


def _mlp(text_emb, image_emb, w1a, w1b, b1, w2, b2):
    return pl.pallas_call(
        _mlp_body,
        out_shape=jax.ShapeDtypeStruct((Q, D_OUT), jnp.float32),
    )(text_emb, image_emb, w1a, w1b, b1, w2, b2)


def _insert3(c, ic, v0, v1, v2, i0, i1, i2):
    lt0, lt1, lt2 = c < v0, c < v1, c < v2
    nv0 = jnp.minimum(v0, c)
    nv1 = jnp.where(lt1, jnp.maximum(v0, c), v1)
    nv2 = jnp.where(lt2, jnp.maximum(v1, c), v2)
    ni0 = jnp.where(lt0, ic, i0)
    ni1 = jnp.where(lt1, jnp.where(lt0, i0, ic), i1)
    ni2 = jnp.where(lt2, jnp.where(lt1, i1, ic), i2)
    return nv0, nv1, nv2, ni0, ni1, ni2


def _ksq_body(keys_ref, ksq_ref):
    kb = keys_ref[...]
    ksq_ref[...] = jnp.sum(kb * kb, axis=1, keepdims=True)


def _ksq(keys):
    return pl.pallas_call(
        _ksq_body,
        grid=(N_BLK,),
        in_specs=[pl.BlockSpec((KB, D_OUT), lambda b: (b, 0))],
        out_specs=pl.BlockSpec((KB, 1), lambda b: (b, 0)),
        out_shape=jax.ShapeDtypeStruct((K_TOT, 1), jnp.float32),
    )(keys)


_GRPS = 8
_QG = Q // _GRPS


def _topk_body(keys_ref, proj_ref, out_val_ref, out_idx_ref, s_scr,
               bval, bidx):
    b = pl.program_id(0)

    @pl.when(b == 0)
    def _init():
        bval[...] = jnp.full_like(bval, jnp.inf)
        bidx[...] = jnp.zeros_like(bidx)

    kb = keys_ref[...]
    s = lax.dot_general(kb, proj_ref[...], (((1,), (1,)), ((), ())),
                        preferred_element_type=jnp.float32)
    ksq = jnp.sum(kb * kb, axis=1, keepdims=True)
    s_scr[...] = s + ksq

    base = jax.lax.broadcasted_iota(jnp.int32, (8, Q), 0) + b * KB
    init = (jnp.full((8, Q), jnp.inf, jnp.float32),) * 3 + (
        jnp.zeros((8, Q), jnp.int32),) * 3

    def fold(r, st):
        c = s_scr[pl.ds(pl.multiple_of(r * 8, 8), 8), :]
        ic = base + r * 8
        return _insert3(c, ic, *st)

    v0, v1, v2, i0, i1, i2 = lax.fori_loop(0, R_BLK, fold, init, unroll=2)

    stack = jnp.concatenate([v0, v1, v2], axis=0)
    istack = jnp.concatenate([i0, i1, i2], axis=0)
    for _ in range(K_TOP):
        m = jnp.min(stack, axis=0, keepdims=True)
        cand = jnp.where(stack == m, istack, jnp.int32(2**31 - 1))
        ci = jnp.min(cand, axis=0, keepdims=True)
        stack = jnp.where(istack == ci, jnp.inf, stack)
        w0, w1, w2, j0, j1, j2 = _insert3(
            m, ci, bval[0:1], bval[1:2], bval[2:3],
            bidx[0:1], bidx[1:2], bidx[2:3])
        bval[0:1], bval[1:2], bval[2:3] = w0, w1, w2
        bidx[0:1], bidx[1:2], bidx[2:3] = j0, j1, j2

    @pl.when(b == pl.num_programs(0) - 1)
    def _emit():
        out_val_ref[...] = bval[...]
        out_idx_ref[...] = bidx[...]


def _topk(keys, proj):
    return pl.pallas_call(
        _topk_body,
        grid=(N_BLK,),
        in_specs=[
            pl.BlockSpec((KB, D_OUT), lambda b: (b, 0)),
            pl.BlockSpec((Q, D_OUT), lambda b: (0, 0)),
        ],
        out_specs=[
            pl.BlockSpec((8, Q), lambda b: (0, 0)),
            pl.BlockSpec((8, Q), lambda b: (0, 0)),
        ],
        out_shape=[
            jax.ShapeDtypeStruct((8, Q), jnp.float32),
            jax.ShapeDtypeStruct((8, Q), jnp.int32),
        ],
        scratch_shapes=[
            pltpu.VMEM((KB, Q), jnp.float32),
            pltpu.VMEM((8, Q), jnp.float32),
            pltpu.VMEM((8, Q), jnp.int32),
        ],
    )(keys, proj)


_NC, _NS = 2, 16
_NW = _NC * _NS
_B_TOT = Q * K_TOP
_BPW = _B_TOT // _NW


def _sc_gather_body(table_hbm, idx_hbm, out_hbm, idx_v, rows_v, sem):
    wid = lax.axis_index("s") * _NC + lax.axis_index("c")
    base = wid * _BPW
    pltpu.sync_copy(idx_hbm.at[pl.ds(base, _BPW)], idx_v)
    pltpu.async_copy(table_hbm.at[idx_v], rows_v, sem).wait()
    pltpu.sync_copy(rows_v, out_hbm.at[pl.ds(base, _BPW)])


def _sc_gather(keys, flat_idx):
    mesh = plsc.VectorSubcoreMesh(core_axis_name="c", subcore_axis_name="s")
    k = functools.partial(
        pl.kernel, mesh=mesh,
        out_type=jax.ShapeDtypeStruct((_B_TOT, D_OUT), jnp.float32),
        scratch_types=[
            pltpu.VMEM((_BPW,), jnp.int32),
            pltpu.VMEM((_BPW, D_OUT), jnp.float32),
            pltpu.SemaphoreType.DMA,
        ],
    )(_sc_gather_body)
    return k(keys, flat_idx)


def kernel(text_emb, image_emb, keys, W1, b1, W2, b2):
    proj = _mlp(text_emb, image_emb, W1[:D_HID], W1[D_HID:],
                b1.reshape(1, D_HID), W2, b2.reshape(1, D_OUT))
    _, idx8 = _topk(keys, proj)
    I = idx8[:K_TOP].T
    retrieved = _sc_gather(keys, I.reshape(-1)).reshape(Q, K_TOP, D_OUT)
    return retrieved, I


# speedup vs baseline: 1.4426x; 1.0175x over previous
import functools

import jax
import jax.numpy as jnp
from jax import lax
from jax.experimental import pallas as pl
from jax.experimental.pallas import tpu as pltpu
from jax.experimental.pallas import tpu_sc as plsc

Q = 1024
D_CAT = 1536
D_HID = 768
D_OUT = 384
K_TOT = 100000
K_TOP = 3
KB = 4000
N_BLK = K_TOT // KB
R_BLK = KB // 8


def _mlp_body(text_ref, image_ref, w1a_ref, w1b_ref, b1_ref, w2_ref, b2_ref,
              proj_ref):
    h = jnp.dot(text_ref[...], w1a_ref[...], preferred_element_type=jnp.float32)
    h = h + jnp.dot(image_ref[...], w1b_ref[...],
                    preferred_element_type=jnp.float32)
    h = jnp.maximum(h + b1_ref[...], 0.0)
    proj_ref[...] = -2.0 * (
        jnp.dot(h, w2_ref[...], preferred_element_type=jnp.float32)
        + b2_ref[...])


def _mlp(text_emb, image_emb, w1a, w1b, b1, w2, b2):
    return pl.pallas_call(
        _mlp_body,
        out_shape=jax.ShapeDtypeStruct((Q, D_OUT), jnp.float32),
    )(text_emb, image_emb, w1a, w1b, b1, w2, b2)


def _insert3(c, ic, v0, v1, v2, i0, i1, i2):
    lt0, lt1, lt2 = c < v0, c < v1, c < v2
    nv0 = jnp.minimum(v0, c)
    nv1 = jnp.where(lt1, jnp.maximum(v0, c), v1)
    nv2 = jnp.where(lt2, jnp.maximum(v1, c), v2)
    ni0 = jnp.where(lt0, ic, i0)
    ni1 = jnp.where(lt1, jnp.where(lt0, i0, ic), i1)
    ni2 = jnp.where(lt2, jnp.where(lt1, i1, ic), i2)
    return nv0, nv1, nv2, ni0, ni1, ni2


def _ksq_body(keys_ref, ksq_ref):
    kb = keys_ref[...]
    ksq_ref[...] = jnp.sum(kb * kb, axis=1, keepdims=True)


def _ksq(keys):
    return pl.pallas_call(
        _ksq_body,
        grid=(N_BLK,),
        in_specs=[pl.BlockSpec((KB, D_OUT), lambda b: (b, 0))],
        out_specs=pl.BlockSpec((KB, 1), lambda b: (b, 0)),
        out_shape=jax.ShapeDtypeStruct((K_TOT, 1), jnp.float32),
    )(keys)


_GRPS = 8
_QG = Q // _GRPS


def _topk_body(keys_ref, proj_ref, out_val_ref, out_idx_ref, s_scr,
               bval, bidx):
    b = pl.program_id(0)

    @pl.when(b == 0)
    def _init():
        bval[...] = jnp.full_like(bval, jnp.inf)
        bidx[...] = jnp.zeros_like(bidx)

    kb = keys_ref[...]
    s = lax.dot_general(kb, proj_ref[...], (((1,), (1,)), ((), ())),
                        preferred_element_type=jnp.float32)
    ksq = jnp.sum(kb * kb, axis=1, keepdims=True)
    s_scr[...] = s + ksq

    base = jax.lax.broadcasted_iota(jnp.int32, (8, Q), 0) + b * KB
    init = (jnp.full((8, Q), jnp.inf, jnp.float32),) * 3 + (
        jnp.zeros((8, Q), jnp.int32),) * 3

    def fold(r, st):
        c = s_scr[pl.ds(pl.multiple_of(r * 8, 8), 8), :]
        ic = base + r * 8
        return _insert3(c, ic, *st)

    v0, v1, v2, i0, i1, i2 = lax.fori_loop(0, R_BLK, fold, init, unroll=2)

    stack = jnp.concatenate([v0, v1, v2], axis=0)
    istack = jnp.concatenate([i0, i1, i2], axis=0)
    for _ in range(K_TOP):
        m = jnp.min(stack, axis=0, keepdims=True)
        cand = jnp.where(stack == m, istack, jnp.int32(2**31 - 1))
        ci = jnp.min(cand, axis=0, keepdims=True)
        stack = jnp.where(istack == ci, jnp.inf, stack)
        w0, w1, w2, j0, j1, j2 = _insert3(
            m, ci, bval[0:1], bval[1:2], bval[2:3],
            bidx[0:1], bidx[1:2], bidx[2:3])
        bval[0:1], bval[1:2], bval[2:3] = w0, w1, w2
        bidx[0:1], bidx[1:2], bidx[2:3] = j0, j1, j2

    @pl.when(b == pl.num_programs(0) - 1)
    def _emit():
        out_val_ref[...] = bval[...]
        out_idx_ref[...] = bidx[...]


def _topk(keys, proj):
    return pl.pallas_call(
        _topk_body,
        grid=(N_BLK,),
        in_specs=[
            pl.BlockSpec((KB, D_OUT), lambda b: (b, 0)),
            pl.BlockSpec((Q, D_OUT), lambda b: (0, 0)),
        ],
        out_specs=[
            pl.BlockSpec((8, Q), lambda b: (0, 0)),
            pl.BlockSpec((8, Q), lambda b: (0, 0)),
        ],
        out_shape=[
            jax.ShapeDtypeStruct((8, Q), jnp.float32),
            jax.ShapeDtypeStruct((8, Q), jnp.int32),
        ],
        scratch_shapes=[
            pltpu.VMEM((KB, Q), jnp.float32),
            pltpu.VMEM((8, Q), jnp.float32),
            pltpu.VMEM((8, Q), jnp.int32),
        ],
    )(keys, proj)


_NC, _NS = 2, 16
_NW = _NC * _NS
_B_TOT = Q * K_TOP
_BPW = _B_TOT // _NW


def _sc_gather_body(table_hbm, idx_hbm, out_hbm, idx_v, rows_v, sem):
    wid = lax.axis_index("s") * _NC + lax.axis_index("c")
    base = wid * _BPW
    pltpu.sync_copy(idx_hbm.at[pl.ds(base, _BPW)], idx_v)
    pltpu.async_copy(table_hbm.at[idx_v], rows_v, sem).wait()
    pltpu.sync_copy(rows_v, out_hbm.at[pl.ds(base, _BPW)])


def _sc_gather(keys, flat_idx):
    mesh = plsc.VectorSubcoreMesh(core_axis_name="c", subcore_axis_name="s")
    k = functools.partial(
        pl.kernel, mesh=mesh,
        out_type=jax.ShapeDtypeStruct((_B_TOT, D_OUT), jnp.float32),
        scratch_types=[
            pltpu.VMEM((_BPW,), jnp.int32),
            pltpu.VMEM((_BPW, D_OUT), jnp.float32),
            pltpu.SemaphoreType.DMA,
        ],
    )(_sc_gather_body)
    return k(keys, flat_idx)


def kernel(text_emb, image_emb, keys, W1, b1, W2, b2):
    proj = _mlp(text_emb, image_emb, W1[:D_HID], W1[D_HID:],
                b1.reshape(1, D_HID), W2, b2.reshape(1, D_OUT))
    _, idx8 = _topk(keys, proj)
    I = idx8[:K_TOP].T
    retrieved = _sc_gather(keys, I.reshape(-1)).reshape(Q, K_TOP, D_OUT)
    return retrieved, I


# speedup vs baseline: 1.4683x; 1.0178x over previous
import functools

import jax
import jax.numpy as jnp
from jax import lax
from jax.experimental import pallas as pl
from jax.experimental.pallas import tpu as pltpu
from jax.experimental.pallas import tpu_sc as plsc

Q = 1024
D_CAT = 1536
D_HID = 768
D_OUT = 384
K_TOT = 100000
K_TOP = 3
KB = 5000
N_BLK = K_TOT // KB
R_BLK = KB // 8


def _mlp_body(text_ref, image_ref, w1a_ref, w1b_ref, b1_ref, w2_ref, b2_ref,
              proj_ref):
    h = jnp.dot(text_ref[...], w1a_ref[...], preferred_element_type=jnp.float32)
    h = h + jnp.dot(image_ref[...], w1b_ref[...],
                    preferred_element_type=jnp.float32)
    h = jnp.maximum(h + b1_ref[...], 0.0)
    proj_ref[...] = -2.0 * (
        jnp.dot(h, w2_ref[...], preferred_element_type=jnp.float32)
        + b2_ref[...])


def _mlp(text_emb, image_emb, w1a, w1b, b1, w2, b2):
    return pl.pallas_call(
        _mlp_body,
        out_shape=jax.ShapeDtypeStruct((Q, D_OUT), jnp.float32),
    )(text_emb, image_emb, w1a, w1b, b1, w2, b2)


def _insert3(c, ic, v0, v1, v2, i0, i1, i2):
    lt0, lt1, lt2 = c < v0, c < v1, c < v2
    nv0 = jnp.minimum(v0, c)
    nv1 = jnp.where(lt1, jnp.maximum(v0, c), v1)
    nv2 = jnp.where(lt2, jnp.maximum(v1, c), v2)
    ni0 = jnp.where(lt0, ic, i0)
    ni1 = jnp.where(lt1, jnp.where(lt0, i0, ic), i1)
    ni2 = jnp.where(lt2, jnp.where(lt1, i1, ic), i2)
    return nv0, nv1, nv2, ni0, ni1, ni2


def _ksq_body(keys_ref, ksq_ref):
    kb = keys_ref[...]
    ksq_ref[...] = jnp.sum(kb * kb, axis=1, keepdims=True)


def _ksq(keys):
    return pl.pallas_call(
        _ksq_body,
        grid=(N_BLK,),
        in_specs=[pl.BlockSpec((KB, D_OUT), lambda b: (b, 0))],
        out_specs=pl.BlockSpec((KB, 1), lambda b: (b, 0)),
        out_shape=jax.ShapeDtypeStruct((K_TOT, 1), jnp.float32),
    )(keys)


_GRPS = 8
_QG = Q // _GRPS


def _topk_body(keys_ref, proj_ref, out_val_ref, out_idx_ref, s_scr,
               bval, bidx):
    b = pl.program_id(0)

    @pl.when(b == 0)
    def _init():
        bval[...] = jnp.full_like(bval, jnp.inf)
        bidx[...] = jnp.zeros_like(bidx)

    kb = keys_ref[...]
    s = lax.dot_general(kb, proj_ref[...], (((1,), (1,)), ((), ())),
                        preferred_element_type=jnp.float32)
    ksq = jnp.sum(kb * kb, axis=1, keepdims=True)
    s_scr[...] = s + ksq

    base = jax.lax.broadcasted_iota(jnp.int32, (8, Q), 0) + b * KB
    init = (jnp.full((8, Q), jnp.inf, jnp.float32),) * 3 + (
        jnp.zeros((8, Q), jnp.int32),) * 3

    def fold(r, st):
        c = s_scr[pl.ds(pl.multiple_of(r * 8, 8), 8), :]
        ic = base + r * 8
        return _insert3(c, ic, *st)

    v0, v1, v2, i0, i1, i2 = lax.fori_loop(0, R_BLK, fold, init, unroll=2)

    stack = jnp.concatenate([v0, v1, v2], axis=0)
    istack = jnp.concatenate([i0, i1, i2], axis=0)
    for _ in range(K_TOP):
        m = jnp.min(stack, axis=0, keepdims=True)
        cand = jnp.where(stack == m, istack, jnp.int32(2**31 - 1))
        ci = jnp.min(cand, axis=0, keepdims=True)
        stack = jnp.where(istack == ci, jnp.inf, stack)
        w0, w1, w2, j0, j1, j2 = _insert3(
            m, ci, bval[0:1], bval[1:2], bval[2:3],
            bidx[0:1], bidx[1:2], bidx[2:3])
        bval[0:1], bval[1:2], bval[2:3] = w0, w1, w2
        bidx[0:1], bidx[1:2], bidx[2:3] = j0, j1, j2

    @pl.when(b == pl.num_programs(0) - 1)
    def _emit():
        out_val_ref[...] = bval[...]
        out_idx_ref[...] = bidx[...]


def _topk(keys, proj):
    return pl.pallas_call(
        _topk_body,
        grid=(N_BLK,),
        in_specs=[
            pl.BlockSpec((KB, D_OUT), lambda b: (b, 0)),
            pl.BlockSpec((Q, D_OUT), lambda b: (0, 0)),
        ],
        out_specs=[
            pl.BlockSpec((8, Q), lambda b: (0, 0)),
            pl.BlockSpec((8, Q), lambda b: (0, 0)),
        ],
        out_shape=[
            jax.ShapeDtypeStruct((8, Q), jnp.float32),
            jax.ShapeDtypeStruct((8, Q), jnp.int32),
        ],
        scratch_shapes=[
            pltpu.VMEM((KB, Q), jnp.float32),
            pltpu.VMEM((8, Q), jnp.float32),
            pltpu.VMEM((8, Q), jnp.int32),
        ],
    )(keys, proj)


_NC, _NS = 2, 16
_NW = _NC * _NS
_B_TOT = Q * K_TOP
_BPW = _B_TOT // _NW


def _sc_gather_body(table_hbm, idx_hbm, out_hbm, idx_v, rows_v, sem):
    wid = lax.axis_index("s") * _NC + lax.axis_index("c")
    base = wid * _BPW
    pltpu.sync_copy(idx_hbm.at[pl.ds(base, _BPW)], idx_v)
    pltpu.async_copy(table_hbm.at[idx_v], rows_v, sem).wait()
    pltpu.sync_copy(rows_v, out_hbm.at[pl.ds(base, _BPW)])


def _sc_gather(keys, flat_idx):
    mesh = plsc.VectorSubcoreMesh(core_axis_name="c", subcore_axis_name="s")
    k = functools.partial(
        pl.kernel, mesh=mesh,
        out_type=jax.ShapeDtypeStruct((_B_TOT, D_OUT), jnp.float32),
        scratch_types=[
            pltpu.VMEM((_BPW,), jnp.int32),
            pltpu.VMEM((_BPW, D_OUT), jnp.float32),
            pltpu.SemaphoreType.DMA,
        ],
    )(_sc_gather_body)
    return k(keys, flat_idx)


def kernel(text_emb, image_emb, keys, W1, b1, W2, b2):
    proj = _mlp(text_emb, image_emb, W1[:D_HID], W1[D_HID:],
                b1.reshape(1, D_HID), W2, b2.reshape(1, D_OUT))
    _, idx8 = _topk(keys, proj)
    I = idx8[:K_TOP].T
    retrieved = _sc_gather(keys, I.reshape(-1)).reshape(Q, K_TOP, D_OUT)
    return retrieved, I


# speedup vs baseline: 1.5826x; 1.0778x over previous
import functools

import jax
import jax.numpy as jnp
from jax import lax
from jax.experimental import pallas as pl
from jax.experimental.pallas import tpu as pltpu
from jax.experimental.pallas import tpu_sc as plsc

Q = 1024
D_CAT = 1536
D_HID = 768
D_OUT = 384
K_TOT = 100000
K_TOP = 3
KB = 5000
N_BLK = K_TOT // KB
R_BLK = KB // 8


def _mlp_body(text_ref, image_ref, w1a_ref, w1b_ref, b1_ref, w2_ref, b2_ref,
              proj_ref):
    h = jnp.dot(text_ref[...], w1a_ref[...], preferred_element_type=jnp.float32)
    h = h + jnp.dot(image_ref[...], w1b_ref[...],
                    preferred_element_type=jnp.float32)
    h = jnp.maximum(h + b1_ref[...], 0.0)
    proj_ref[...] = -2.0 * (
        jnp.dot(h, w2_ref[...], preferred_element_type=jnp.float32)
        + b2_ref[...])


def _mlp(text_emb, image_emb, w1a, w1b, b1, w2, b2):
    return pl.pallas_call(
        _mlp_body,
        out_shape=jax.ShapeDtypeStruct((Q, D_OUT), jnp.float32),
    )(text_emb, image_emb, w1a, w1b, b1, w2, b2)


def _insert3(c, ic, v0, v1, v2, i0, i1, i2):
    lt0, lt1, lt2 = c < v0, c < v1, c < v2
    nv0 = jnp.minimum(v0, c)
    nv1 = jnp.where(lt1, jnp.maximum(v0, c), v1)
    nv2 = jnp.where(lt2, jnp.maximum(v1, c), v2)
    ni0 = jnp.where(lt0, ic, i0)
    ni1 = jnp.where(lt1, jnp.where(lt0, i0, ic), i1)
    ni2 = jnp.where(lt2, jnp.where(lt1, i1, ic), i2)
    return nv0, nv1, nv2, ni0, ni1, ni2


def _ksq_body(keys_ref, ksq_ref):
    kb = keys_ref[...]
    ksq_ref[...] = jnp.sum(kb * kb, axis=1, keepdims=True)


def _ksq(keys):
    return pl.pallas_call(
        _ksq_body,
        grid=(N_BLK,),
        in_specs=[pl.BlockSpec((KB, D_OUT), lambda b: (b, 0))],
        out_specs=pl.BlockSpec((KB, 1), lambda b: (b, 0)),
        out_shape=jax.ShapeDtypeStruct((K_TOT, 1), jnp.float32),
    )(keys)


_GRPS = 8
_QG = Q // _GRPS


def _topk_body(keys_ref, proj_ref, out_val_ref, out_idx_ref, s_scr,
               bval, bidx):
    b = pl.program_id(0)

    @pl.when(b == 0)
    def _init():
        bval[...] = jnp.full_like(bval, jnp.inf)
        bidx[...] = jnp.zeros_like(bidx)

    kb = keys_ref[...]
    s = lax.dot_general(kb, proj_ref[...], (((1,), (1,)), ((), ())),
                        preferred_element_type=jnp.float32)
    ksq = jnp.sum(kb * kb, axis=1, keepdims=True)
    s_scr[...] = s + ksq

    base = jax.lax.broadcasted_iota(jnp.int32, (8, Q), 0) + b * KB
    init = (jnp.full((8, Q), jnp.inf, jnp.float32),) * 3 + (
        jnp.zeros((8, Q), jnp.int32),) * 3

    def fold(r, st):
        c = s_scr[pl.ds(pl.multiple_of(r * 8, 8), 8), :]
        ic = base + r * 8
        return _insert3(c, ic, *st)

    v0, v1, v2, i0, i1, i2 = lax.fori_loop(0, R_BLK, fold, init, unroll=4)

    stack = jnp.concatenate([v0, v1, v2], axis=0)
    istack = jnp.concatenate([i0, i1, i2], axis=0)
    for _ in range(K_TOP):
        m = jnp.min(stack, axis=0, keepdims=True)
        cand = jnp.where(stack == m, istack, jnp.int32(2**31 - 1))
        ci = jnp.min(cand, axis=0, keepdims=True)
        stack = jnp.where(istack == ci, jnp.inf, stack)
        w0, w1, w2, j0, j1, j2 = _insert3(
            m, ci, bval[0:1], bval[1:2], bval[2:3],
            bidx[0:1], bidx[1:2], bidx[2:3])
        bval[0:1], bval[1:2], bval[2:3] = w0, w1, w2
        bidx[0:1], bidx[1:2], bidx[2:3] = j0, j1, j2

    @pl.when(b == pl.num_programs(0) - 1)
    def _emit():
        out_val_ref[...] = bval[...]
        out_idx_ref[...] = bidx[...]


def _topk(keys, proj):
    return pl.pallas_call(
        _topk_body,
        grid=(N_BLK,),
        in_specs=[
            pl.BlockSpec((KB, D_OUT), lambda b: (b, 0)),
            pl.BlockSpec((Q, D_OUT), lambda b: (0, 0)),
        ],
        out_specs=[
            pl.BlockSpec((8, Q), lambda b: (0, 0)),
            pl.BlockSpec((8, Q), lambda b: (0, 0)),
        ],
        out_shape=[
            jax.ShapeDtypeStruct((8, Q), jnp.float32),
            jax.ShapeDtypeStruct((8, Q), jnp.int32),
        ],
        scratch_shapes=[
            pltpu.VMEM((KB, Q), jnp.float32),
            pltpu.VMEM((8, Q), jnp.float32),
            pltpu.VMEM((8, Q), jnp.int32),
        ],
    )(keys, proj)


_NC, _NS = 2, 16
_NW = _NC * _NS
_B_TOT = Q * K_TOP
_BPW = _B_TOT // _NW


def _sc_gather_body(table_hbm, idx_hbm, out_hbm, idx_v, rows_v, sem):
    wid = lax.axis_index("s") * _NC + lax.axis_index("c")
    base = wid * _BPW
    pltpu.sync_copy(idx_hbm.at[pl.ds(base, _BPW)], idx_v)
    pltpu.async_copy(table_hbm.at[idx_v], rows_v, sem).wait()
    pltpu.sync_copy(rows_v, out_hbm.at[pl.ds(base, _BPW)])


def _sc_gather(keys, flat_idx):
    mesh = plsc.VectorSubcoreMesh(core_axis_name="c", subcore_axis_name="s")
    k = functools.partial(
        pl.kernel, mesh=mesh,
        out_type=jax.ShapeDtypeStruct((_B_TOT, D_OUT), jnp.float32),
        scratch_types=[
            pltpu.VMEM((_BPW,), jnp.int32),
            pltpu.VMEM((_BPW, D_OUT), jnp.float32),
            pltpu.SemaphoreType.DMA,
        ],
    )(_sc_gather_body)
    return k(keys, flat_idx)


def kernel(text_emb, image_emb, keys, W1, b1, W2, b2):
    proj = _mlp(text_emb, image_emb, W1[:D_HID], W1[D_HID:],
                b1.reshape(1, D_HID), W2, b2.reshape(1, D_OUT))
    _, idx8 = _topk(keys, proj)
    I = idx8[:K_TOP].T
    retrieved = _sc_gather(keys, I.reshape(-1)).reshape(Q, K_TOP, D_OUT)
    return retrieved, I
